# baseline (speedup 1.0000x reference)
import functools

import jax
import jax.numpy as jnp
from jax import lax
from jax.experimental import pallas as pl
from jax.experimental.pallas import tpu as pltpu
from jax.experimental.pallas import tpu_sc as plsc

B = 16384
L = 200
V = 1000
D = 64

_NC, _NS = 2, 16
_NW = _NC * _NS
_S = B // _NW
_G = 32
_NCHUNK = _S // _G
_NGRP = L // 16
_AW = (_NGRP + 1) * 16


_PUNROLL = 8
_BW = 128
_NBLK = _S // _BW
_SUBS = _BW // _G


def _hist_body(half, nslice, xt_hbm, c_hbm,
               idxA, idxB, cnt0, cnt1,
               siA, siB, so0, so1):
    sw = _S // nslice
    wid = lax.axis_index("s") * _NC + lax.axis_index("c")
    base = half * (B // nslice) + wid * sw
    obase = wid * sw

    idx_j, si_j = (idxA, idxB), (siA, siB)
    cnt_b, so_b = (cnt0, cnt1), (so0, so1)

    ones = jnp.full((16,), 1.0, jnp.float32)
    zeros = jnp.full((16,), 0.0, jnp.float32)
    row_lo = lax.iota(jnp.int32, 16)
    row_hi = row_lo + 16

    def _start_blk(j):
        col0 = base + j * _BW
        pltpu.async_copy(xt_hbm.at[:, pl.ds(col0, _BW)], idx_j[j % 2],
                         si_j[j % 2])

    def _wait_blk(j):
        pltpu.make_async_copy(xt_hbm.at[:, pl.ds(base, _BW)], idx_j[j % 2],
                              si_j[j % 2]).wait()

    def _start_out(c):
        row0 = obase + c * _G
        pltpu.async_copy(cnt_b[c % 2], c_hbm.at[pl.ds(row0, _G)],
                         so_b[c % 2])

    def _wait_out(b):
        pltpu.make_async_copy(cnt_b[b], c_hbm.at[pl.ds(obase, _G)],
                              so_b[b]).wait()

    def _sweep(c):
        cnt_v = cnt_b[c % 2]
        idx_v = idx_j[(c // _SUBS) % 2]
        off = (c % _SUBS) * _G

        def _pos(g, _c):
            l0 = g * _PUNROLL
            vecs = []
            for p in range(_PUNROLL):
                vecs.append(idx_v[l0 + p, pl.ds(off, 16)])
                vecs.append(idx_v[l0 + p, pl.ds(off + 16, 16)])
            for p in range(_PUNROLL):
                plsc.addupdate_scatter(cnt_v, [row_lo, vecs[2 * p]], ones)
                plsc.addupdate_scatter(cnt_v, [row_hi, vecs[2 * p + 1]], ones)
            return _c
        lax.fori_loop(0, L // _PUNROLL, _pos, None)

    def _clear_chunk(b):
        cnt_v = cnt_b[b]

        def _clear(i, _):
            for j in range(V // 16):
                cnt_v[i, pl.ds(j * 16, 16)] = zeros
            cnt_v[i, pl.ds(V - 16, 16)] = zeros
            return _
        lax.fori_loop(0, _G, _clear, None)

    nblk = sw // _BW
    nsub = sw // _G
    _start_blk(0)
    if nblk > 1:
        _start_blk(1)
    for c in range(nsub):
        if c >= 2:
            _wait_out(c % 2)
        _clear_chunk(c % 2)
        if c % _SUBS == 0:
            _wait_blk(c // _SUBS)
        _sweep(c)
        if c % _SUBS == _SUBS - 1 and (c // _SUBS) + 2 < nblk:
            _start_blk((c // _SUBS) + 2)
        _start_out(c)

    for b in (0, 1):
        _wait_out(b)


@functools.cache
def _hist(half, nslice):
    return functools.partial(
        pl.kernel,
        mesh=plsc.VectorSubcoreMesh(core_axis_name="c", subcore_axis_name="s"),
        out_type=jax.ShapeDtypeStruct((B // nslice, V), jnp.float32),
        scratch_types=[
            pltpu.VMEM((L, _BW), jnp.int32),
            pltpu.VMEM((L, _BW), jnp.int32),
            pltpu.VMEM((_G, V), jnp.float32),
            pltpu.VMEM((_G, V), jnp.float32),
            pltpu.SemaphoreType.DMA,
            pltpu.SemaphoreType.DMA,
            pltpu.SemaphoreType.DMA,
            pltpu.SemaphoreType.DMA,
        ],
        compiler_params=pltpu.CompilerParams(needs_layout_passes=False),
    )(functools.partial(_hist_body, half, nslice))


_BLK = 2048


def _tc_body(c_ref, emb_ref, w_ref, b_ref, o_ref):
    m = jnp.dot(c_ref[...], emb_ref[...], preferred_element_type=jnp.float32)
    r = jnp.maximum(m * (1.0 / L), 0.0)
    yt = lax.dot_general(w_ref[...], r, (((1,), (1,)), ((), ())),
                         preferred_element_type=jnp.float32)
    o_ref[...] = jax.nn.sigmoid(yt + b_ref[...])


def _tc_slice_body(c_ref, emb_ref, w_ref, b_ref, y_ref, o_ref):
    del y_ref
    _tc_body(c_ref, emb_ref, w_ref, b_ref, o_ref)


@functools.cache
def _tc(h, nslice):
    nblk = B // nslice // _BLK
    specs = [
        pl.BlockSpec((_BLK, V), lambda i: (i, 0)),
        pl.BlockSpec((V, D), lambda i: (0, 0)),
        pl.BlockSpec((V, D), lambda i: (0, 0)),
        pl.BlockSpec((V, 1), lambda i: (0, 0)),
    ]
    off = h * nblk
    out_spec = pl.BlockSpec((V, _BLK), lambda i: (0, off + i))
    if h == 0:
        return pl.pallas_call(
            _tc_body,
            grid=(nblk,),
            in_specs=specs,
            out_specs=out_spec,
            out_shape=jax.ShapeDtypeStruct((V, B), jnp.float32),
        )
    return pl.pallas_call(
        _tc_slice_body,
        grid=(nblk,),
        in_specs=specs + [pl.BlockSpec(memory_space=pl.ANY)],
        out_specs=out_spec,
        out_shape=jax.ShapeDtypeStruct((V, B), jnp.float32),
        input_output_aliases={4: 0},
    )


_NSLICE = 4


def kernel(x, emb, W, b):
    xt = x.astype(jnp.int32).T
    b2 = b.reshape(V, 1)
    counts = [_hist(h, _NSLICE)(xt) for h in range(_NSLICE)]
    yt = _tc(0, _NSLICE)(counts[0], emb, W, b2)
    for h in range(1, _NSLICE):
        yt = _tc(h, _NSLICE)(counts[h], emb, W, b2, yt)
    return yt.T


# speedup vs baseline: 1.0033x; 1.0033x over previous
import functools

import jax
import jax.numpy as jnp
from jax import lax
from jax.experimental import pallas as pl
from jax.experimental.pallas import tpu as pltpu
from jax.experimental.pallas import tpu_sc as plsc

B = 16384
L = 200
V = 1000
D = 64

_NC, _NS = 2, 16
_NW = _NC * _NS
_S = B // _NW
_G = 32
_NCHUNK = _S // _G
_NGRP = L // 16
_AW = (_NGRP + 1) * 16


_PUNROLL = 8
_BW = 128
_NBLK = _S // _BW
_SUBS = _BW // _G


def _hist_body(half, nslice, xt_hbm, c_hbm,
               idxA, idxB, cnt0, cnt1,
               siA, siB, so0, so1):
    sw = _S // nslice
    wid = lax.axis_index("s") * _NC + lax.axis_index("c")
    base = half * (B // nslice) + wid * sw
    obase = wid * sw

    idx_j, si_j = (idxA, idxB), (siA, siB)
    cnt_b, so_b = (cnt0, cnt1), (so0, so1)

    ones = jnp.full((16,), 1.0, jnp.float32)
    zeros = jnp.full((16,), 0.0, jnp.float32)
    row_lo = lax.iota(jnp.int32, 16)
    row_hi = row_lo + 16

    def _start_blk(j):
        col0 = base + j * _BW
        pltpu.async_copy(xt_hbm.at[:, pl.ds(col0, _BW)], idx_j[j % 2],
                         si_j[j % 2])

    def _wait_blk(j):
        pltpu.make_async_copy(xt_hbm.at[:, pl.ds(base, _BW)], idx_j[j % 2],
                              si_j[j % 2]).wait()

    def _start_out(c):
        row0 = obase + c * _G
        pltpu.async_copy(cnt_b[c % 2], c_hbm.at[pl.ds(row0, _G)],
                         so_b[c % 2])

    def _wait_out(b):
        pltpu.make_async_copy(cnt_b[b], c_hbm.at[pl.ds(obase, _G)],
                              so_b[b]).wait()

    def _sweep(c):
        cnt_v = cnt_b[c % 2]
        idx_v = idx_j[(c // _SUBS) % 2]
        off = (c % _SUBS) * _G

        def _pos(g, _c):
            l0 = g * _PUNROLL
            vecs = []
            for p in range(_PUNROLL):
                vecs.append(idx_v[l0 + p, pl.ds(off, 16)])
                vecs.append(idx_v[l0 + p, pl.ds(off + 16, 16)])
            for p in range(_PUNROLL):
                plsc.addupdate_scatter(cnt_v, [row_lo, vecs[2 * p]], ones)
                plsc.addupdate_scatter(cnt_v, [row_hi, vecs[2 * p + 1]], ones)
            return _c
        lax.fori_loop(0, L // _PUNROLL, _pos, None)

    def _clear_chunk(b):
        cnt_v = cnt_b[b]

        def _clear(i, _):
            for j in range(V // 16):
                cnt_v[i, pl.ds(j * 16, 16)] = zeros
            cnt_v[i, pl.ds(V - 16, 16)] = zeros
            return _
        lax.fori_loop(0, _G, _clear, None)

    nblk = sw // _BW
    nsub = sw // _G
    _start_blk(0)
    if nblk > 1:
        _start_blk(1)
    for c in range(nsub):
        if c >= 2:
            _wait_out(c % 2)
        _clear_chunk(c % 2)
        if c % _SUBS == 0:
            _wait_blk(c // _SUBS)
        _sweep(c)
        if c % _SUBS == _SUBS - 1 and (c // _SUBS) + 2 < nblk:
            _start_blk((c // _SUBS) + 2)
        _start_out(c)

    for b in (0, 1):
        _wait_out(b)


@functools.cache
def _hist(half, nslice):
    return functools.partial(
        pl.kernel,
        mesh=plsc.VectorSubcoreMesh(core_axis_name="c", subcore_axis_name="s"),
        out_type=jax.ShapeDtypeStruct((B // nslice, V), jnp.float32),
        scratch_types=[
            pltpu.VMEM((L, _BW), jnp.int32),
            pltpu.VMEM((L, _BW), jnp.int32),
            pltpu.VMEM((_G, V), jnp.float32),
            pltpu.VMEM((_G, V), jnp.float32),
            pltpu.SemaphoreType.DMA,
            pltpu.SemaphoreType.DMA,
            pltpu.SemaphoreType.DMA,
            pltpu.SemaphoreType.DMA,
        ],
        compiler_params=pltpu.CompilerParams(needs_layout_passes=False),
    )(functools.partial(_hist_body, half, nslice))


_BLK = 2048


def _tc_body(c_ref, emb_ref, w_ref, b_ref, o_ref):
    m = jnp.dot(c_ref[...], emb_ref[...], preferred_element_type=jnp.float32)
    r = jnp.maximum(m * (1.0 / L), 0.0)
    yt = lax.dot_general(w_ref[...], r, (((1,), (1,)), ((), ())),
                         preferred_element_type=jnp.float32)
    o_ref[...] = jax.nn.sigmoid(yt + b_ref[...])


def _tc_slice_body(c_ref, emb_ref, w_ref, b_ref, y_ref, o_ref):
    del y_ref
    _tc_body(c_ref, emb_ref, w_ref, b_ref, o_ref)


@functools.cache
def _tc(h, nslice):
    nblk = B // nslice // _BLK
    specs = [
        pl.BlockSpec((_BLK, V), lambda i: (i, 0)),
        pl.BlockSpec((V, D), lambda i: (0, 0)),
        pl.BlockSpec((V, D), lambda i: (0, 0)),
        pl.BlockSpec((V, 1), lambda i: (0, 0)),
    ]
    off = h * nblk
    out_spec = pl.BlockSpec((V, _BLK), lambda i: (0, off + i))
    if h == 0:
        return pl.pallas_call(
            _tc_body,
            grid=(nblk,),
            in_specs=specs,
            out_specs=out_spec,
            out_shape=jax.ShapeDtypeStruct((V, B), jnp.float32),
        )
    return pl.pallas_call(
        _tc_slice_body,
        grid=(nblk,),
        in_specs=specs + [pl.BlockSpec(memory_space=pl.ANY)],
        out_specs=out_spec,
        out_shape=jax.ShapeDtypeStruct((V, B), jnp.float32),
        input_output_aliases={4: 0},
    )


_NSLICE = 2


def kernel(x, emb, W, b):
    xt = x.astype(jnp.int32).T
    b2 = b.reshape(V, 1)
    counts = [_hist(h, _NSLICE)(xt) for h in range(_NSLICE)]
    yt = _tc(0, _NSLICE)(counts[0], emb, W, b2)
    for h in range(1, _NSLICE):
        yt = _tc(h, _NSLICE)(counts[h], emb, W, b2, yt)
    return yt.T
